# baseline (speedup 1.0000x reference)
import jax
import jax.numpy as jnp
from jax.experimental import pallas as pl
from jax.experimental.pallas import tpu as pltpu


def _scatter_kernel(tok_ref, wphi_ref, out_ref):
    n_tok = tok_ref.shape[0]
    rows, cols = out_ref.shape
    vids = (jax.lax.broadcasted_iota(jnp.int32, (rows, cols), 0) * cols
            + jax.lax.broadcasted_iota(jnp.int32, (rows, cols), 1))
    acc = jnp.zeros((rows, cols), jnp.float32)
    for n in range(n_tok):
        acc = acc + jnp.where(vids == tok_ref[n], wphi_ref[n], 0.0)
    out_ref[:, :] = acc


def _bcast_kernel(pen_ref, out_ref):
    out_ref[:, :] = jnp.broadcast_to(pen_ref[:, :], out_ref.shape)


def kernel(level, candidate_logits, tokens, phis, layer_weights):
    B, V = candidate_logits.shape
    pool_tokens = tokens[:, level]
    w = jnp.cumsum(layer_weights)[level]
    wphi = phis * w

    rows = 8
    cols = V // rows
    pen8 = pl.pallas_call(
        _scatter_kernel,
        grid_spec=pltpu.PrefetchScalarGridSpec(
            num_scalar_prefetch=2,
            grid=(1,),
            in_specs=[],
            out_specs=pl.BlockSpec((rows, cols), lambda i, *_: (0, 0)),
        ),
        out_shape=jax.ShapeDtypeStruct((rows, cols), jnp.float32),
    )(pool_tokens, wphi)
    pen = pen8.reshape(1, V)

    BBLK = 256
    VBLK = 2048
    grid = (B // BBLK, pl.cdiv(V, VBLK))
    out = pl.pallas_call(
        _bcast_kernel,
        grid=grid,
        in_specs=[pl.BlockSpec((1, VBLK), lambda b, v: (0, v))],
        out_specs=pl.BlockSpec((BBLK, VBLK), lambda b, v: (b, v)),
        out_shape=jax.ShapeDtypeStruct((B, V), jnp.float32),
    )(pen)
    return out


# speedup vs baseline: 1.0956x; 1.0956x over previous
import jax
import jax.numpy as jnp
from jax.experimental import pallas as pl
from jax.experimental.pallas import tpu as pltpu

RSTRIP = 8


def _scatter_kernel(tok_ref, wphi_ref, out_ref):
    n_tok = tok_ref.shape[0]
    rows, cols = out_ref.shape
    vids = (jax.lax.broadcasted_iota(jnp.int32, (rows, cols), 0) * cols
            + jax.lax.broadcasted_iota(jnp.int32, (rows, cols), 1))
    acc = jnp.zeros((rows, cols), jnp.float32)
    for n in range(n_tok):
        acc = acc + jnp.where(vids == tok_ref[n], wphi_ref[n], 0.0)
    out_ref[:, :] = acc


def _bcast_kernel(pen_ref, out_ref, scratch, sem):
    B = out_ref.shape[0]
    scratch[:, :] = jnp.broadcast_to(pen_ref[:, :], scratch.shape)
    nstrips = B // RSTRIP
    for i in range(nstrips):
        pltpu.make_async_copy(
            scratch, out_ref.at[pl.ds(i * RSTRIP, RSTRIP), :], sem
        ).start()
    for i in range(nstrips):
        pltpu.make_async_copy(
            scratch, out_ref.at[pl.ds(i * RSTRIP, RSTRIP), :], sem
        ).wait()


def kernel(level, candidate_logits, tokens, phis, layer_weights):
    B, V = candidate_logits.shape
    pool_tokens = tokens[:, level]
    w = jnp.cumsum(layer_weights)[level]
    wphi = phis * w

    rows = 8
    cols = V // rows
    pen8 = pl.pallas_call(
        _scatter_kernel,
        grid_spec=pltpu.PrefetchScalarGridSpec(
            num_scalar_prefetch=2,
            grid=(1,),
            in_specs=[],
            out_specs=pl.BlockSpec((rows, cols), lambda i, *_: (0, 0)),
        ),
        out_shape=jax.ShapeDtypeStruct((rows, cols), jnp.float32),
    )(pool_tokens, wphi)
    pen = pen8.reshape(1, V)

    out = pl.pallas_call(
        _bcast_kernel,
        in_specs=[pl.BlockSpec(memory_space=pltpu.MemorySpace.VMEM)],
        out_specs=pl.BlockSpec(memory_space=pltpu.MemorySpace.HBM),
        out_shape=jax.ShapeDtypeStruct((B, V), jnp.float32),
        scratch_shapes=[
            pltpu.VMEM((RSTRIP, V), jnp.float32),
            pltpu.SemaphoreType.DMA,
        ],
    )(pen)
    return out
